# baseline (speedup 1.0000x reference)
import functools

import jax
import jax.numpy as jnp
from jax import lax
from jax.experimental import pallas as pl
from jax.experimental.pallas import tpu as pltpu
from jax.experimental.pallas import tpu_sc as plsc

N_VOCAB = 1000000
D_MODEL = 32
BATCH = 4096
HIST = 200

NC = 2
NS = 16
NW = NC * NS
BPW = BATCH // NW

_mesh = plsc.VectorSubcoreMesh(core_axis_name="c", subcore_axis_name="s")


@functools.partial(
    pl.kernel,
    mesh=_mesh,
    out_type=jax.ShapeDtypeStruct((HIST, 4, NW, 8, 128), jnp.float32),
    scratch_types=[
        pltpu.VMEM((HIST, BPW), jnp.int32),
        pltpu.VMEM((2, BPW, D_MODEL), jnp.float32),
        pltpu.VMEM((2, 4, 8, 128), jnp.float32),
        pltpu.SemaphoreType.DMA,
        pltpu.SemaphoreType.DMA,
        pltpu.SemaphoreType.DMA,
        pltpu.SemaphoreType.DMA,
    ],
    compiler_params=pltpu.CompilerParams(
        use_tc_tiling_on_sc=False,
        needs_layout_passes=False,
        disable_bounds_checks=True,
    ),
)
def _embed_gather(table_hbm, idx_hbm, out_hbm, idx_v, stage_v, trans_v,
                  gsem0, gsem1, wsem0, wsem1):
    wid = lax.axis_index("s") * NC + lax.axis_index("c")
    gsems = (gsem0, gsem1)
    wsems = (wsem0, wsem1)
    iota16 = lax.iota(jnp.int32, 16)

    pltpu.sync_copy(idx_hbm.at[wid], idx_v)

    def issue_gather(h, b):
        pltpu.async_copy(
            table_hbm.at[idx_v.at[h]], stage_v.at[b], gsems[b]
        )

    def wait_gather(b):
        pltpu.make_async_copy(
            table_hbm.at[pl.ds(0, BPW)], stage_v.at[b], gsems[b]
        ).wait()

    zero16 = iota16 * 0

    def transpose_block(b):
        for l in range(BPW // 16):
            i_vec = iota16 + (16 * l)
            for d in range(D_MODEL):
                vals = plsc.load_gather(stage_v.at[b], [i_vec, zero16 + d])
                trans_v[b, d // 8, d % 8, pl.ds(16 * l, 16)] = vals

    def issue_write(h, b):
        pltpu.async_copy(
            trans_v.at[b], out_hbm.at[h, :, wid], wsems[b]
        )

    def wait_write(b):
        pltpu.make_async_copy(
            trans_v.at[b], out_hbm.at[0, :, wid], wsems[b]
        ).wait()

    issue_gather(0, 0)
    issue_gather(1, 1)
    wait_gather(0)
    transpose_block(0)
    issue_write(0, 0)

    def pair_body(p, carry):
        for b in range(2):
            h = 2 * p + b
            wait_write(b)
            issue_gather(h, b)
            wait_gather(1 - b)
            transpose_block(1 - b)
            issue_write(h - 1, 1 - b)
        return carry

    lax.fori_loop(1, HIST // 2, pair_body, 0)

    wait_gather(1)
    transpose_block(1)
    issue_write(HIST - 1, 1)
    wait_write(0)
    wait_write(1)


def kernel(tok_ids, embedding):
    idx = tok_ids.T.astype(jnp.int32).reshape(HIST, NW, BPW).transpose(1, 0, 2)
    out5 = _embed_gather(embedding, idx)
    return jnp.transpose(out5, (2, 4, 0, 1, 3)).reshape(BATCH, HIST, D_MODEL)


# speedup vs baseline: 1.2952x; 1.2952x over previous
import functools

import jax
import jax.numpy as jnp
from jax import lax
from jax.experimental import pallas as pl
from jax.experimental.pallas import tpu as pltpu
from jax.experimental.pallas import tpu_sc as plsc

N_VOCAB = 1000000
D_MODEL = 32
BATCH = 4096
HIST = 200

NC = 2
NS = 16
NW = NC * NS
BPW = BATCH // NW

_mesh = plsc.VectorSubcoreMesh(core_axis_name="c", subcore_axis_name="s")


@functools.partial(
    pl.kernel,
    mesh=_mesh,
    out_type=jax.ShapeDtypeStruct((HIST, 4, NW, 8, 128), jnp.float32),
    scratch_types=[
        pltpu.VMEM((HIST, BPW), jnp.int32),
        pltpu.VMEM((2, BPW, D_MODEL), jnp.float32),
        pltpu.VMEM((2, 4, 8, 128), jnp.float32),
        pltpu.SemaphoreType.DMA,
        pltpu.SemaphoreType.DMA,
        pltpu.SemaphoreType.DMA,
        pltpu.SemaphoreType.DMA,
    ],
    compiler_params=pltpu.CompilerParams(
        use_tc_tiling_on_sc=False,
        needs_layout_passes=False,
        disable_bounds_checks=True,
    ),
)
def _embed_gather(table_hbm, idx_hbm, out_hbm, idx_v, stage_v, trans_v,
                  gsem0, gsem1, wsem0, wsem1):
    wid = lax.axis_index("s") * NC + lax.axis_index("c")
    gsems = (gsem0, gsem1)
    wsems = (wsem0, wsem1)
    iota16 = lax.iota(jnp.int32, 16)

    pltpu.sync_copy(idx_hbm.at[wid], idx_v)

    def issue_gather(h, b):
        pltpu.async_copy(
            table_hbm.at[idx_v.at[h]], stage_v.at[b], gsems[b]
        )

    def wait_gather(b):
        pltpu.make_async_copy(
            table_hbm.at[pl.ds(0, BPW)], stage_v.at[b], gsems[b]
        ).wait()

    zero16 = iota16 * 0

    def transpose_block(b):
        @plsc.parallel_loop(0, BPW // 16, unroll=2)
        def _tl(l):
            i_vec = iota16 + 16 * l
            for d in range(D_MODEL):
                vals = plsc.load_gather(stage_v.at[b], [i_vec, zero16 + d])
                trans_v[b, d // 8, d % 8, pl.ds(16 * l, 16)] = vals

    def issue_write(h, b):
        pltpu.async_copy(
            trans_v.at[b], out_hbm.at[h, :, wid], wsems[b]
        )

    def wait_write(b):
        pltpu.make_async_copy(
            trans_v.at[b], out_hbm.at[0, :, wid], wsems[b]
        ).wait()

    issue_gather(0, 0)
    issue_gather(1, 1)
    wait_gather(0)
    transpose_block(0)
    issue_write(0, 0)

    def pair_body(p, carry):
        for b in range(2):
            h = 2 * p + b
            wait_write(b)
            issue_gather(h, b)
            wait_gather(1 - b)
            transpose_block(1 - b)
            issue_write(h - 1, 1 - b)
        return carry

    lax.fori_loop(1, HIST // 2, pair_body, 0)

    wait_gather(1)
    transpose_block(1)
    issue_write(HIST - 1, 1)
    wait_write(0)
    wait_write(1)


def kernel(tok_ids, embedding):
    idx = tok_ids.T.astype(jnp.int32).reshape(HIST, NW, BPW).transpose(1, 0, 2)
    out5 = _embed_gather(embedding, idx)
    return jnp.transpose(out5, (2, 4, 0, 1, 3)).reshape(BATCH, HIST, D_MODEL)
